# baseline (speedup 1.0000x reference)
import functools

import jax
import jax.numpy as jnp
import numpy as np
from jax import lax
from jax.experimental import pallas as pl
from jax.experimental.pallas import tpu as pltpu
from jax.experimental.pallas import tpu_sc as plsc

W = 32
R_TOTAL = 16384
R_SC = 4096
R_TC = R_TOTAL - R_SC
CHUNK = 16384
SC_BASE = R_TC * 256
SC_ELEMS = R_SC * 256
NCHUNK = SC_ELEMS // (W * CHUNK)
BLK = CHUNK // 64

_SCALE = 3.5
_STEP = 2.0 / 7.0

BR = 2048


def _quantize_chunk(x_v, q_v, f_v):
    lane4 = lax.broadcasted_iota(jnp.int32, (16,), 0) * 4

    @plsc.parallel_loop(0, BLK, 1, unroll=8)
    def blk(i):
        i0 = lane4 + i * 64
        ids = []
        for j in range(4):
            ij = i0 + j
            x = plsc.load_gather(x_v, [ij])
            t = x * _SCALE + 4.0
            t = jnp.minimum(jnp.maximum(t, 0.0), 7.5)
            idx = t.astype(jnp.int32)
            q = idx.astype(jnp.float32) * _STEP - 1.0
            plsc.store_scatter(q_v, [ij], q)
            ids.append(idx)
        flat = ids[0] | (ids[1] << 3) | (ids[2] << 6) | (ids[3] << 9)
        f_v[pl.ds(i * 16, 16)] = flat


def _sc_body(x_hbm, q_hbm, f_hbm,
             x0, x1, q0, q1, f0, f1, si0, si1, so0, so1):
    wid = lax.axis_index("s") * 2 + lax.axis_index("c")
    base = pl.multiple_of(SC_BASE + wid * (NCHUNK * CHUNK), CHUNK)
    xb, qb, fb = [x0, x1], [q0, q1], [f0, f1]
    si, so = [si0, si1], [so0, so1]
    in_copy = [None, None]
    out_q = [None, None]
    out_f = [None, None]

    in_copy[0] = pltpu.async_copy(x_hbm.at[pl.ds(base, CHUNK)], xb[0], si[0])
    for c in range(NCHUNK):
        b = c & 1
        if c + 1 < NCHUNK:
            in_copy[1 - b] = pltpu.async_copy(
                x_hbm.at[pl.ds(base + (c + 1) * CHUNK, CHUNK)],
                xb[1 - b], si[1 - b])
        in_copy[b].wait()
        if c >= 2:
            out_q[b].wait()
            out_f[b].wait()
        _quantize_chunk(xb[b], qb[b], fb[b])
        off = pl.multiple_of(wid * (NCHUNK * CHUNK) + c * CHUNK, CHUNK)
        out_q[b] = pltpu.async_copy(
            qb[b], q_hbm.at[pl.ds(off, CHUNK)], so[b])
        foff = pl.multiple_of(
            wid * (NCHUNK * CHUNK // 4) + c * (CHUNK // 4), CHUNK // 4)
        out_f[b] = pltpu.async_copy(
            fb[b], f_hbm.at[pl.ds(foff, CHUNK // 4)], so[b])
    for b in range(min(2, NCHUNK)):
        out_q[b].wait()
        out_f[b].wait()


@functools.partial(
    pl.kernel,
    out_type=(
        jax.ShapeDtypeStruct((SC_ELEMS,), jnp.float32),
        jax.ShapeDtypeStruct((SC_ELEMS // 4,), jnp.int32),
    ),
    mesh=plsc.VectorSubcoreMesh(core_axis_name="c", subcore_axis_name="s"),
    scratch_types=(
        [pltpu.VMEM((CHUNK,), jnp.float32) for _ in range(4)]
        + [pltpu.VMEM((CHUNK // 4,), jnp.int32) for _ in range(2)]
        + [pltpu.SemaphoreType.DMA for _ in range(4)]
    ),
    compiler_params=pltpu.CompilerParams(needs_layout_passes=False),
)
def _sc_call(x_hbm, q_hbm, f_hbm, *bufs):
    _sc_body(x_hbm, q_hbm, f_hbm, *bufs)


def _tc_body(x_ref, s_ref, q_ref, f_ref):
    x = x_ref[...]
    t = jnp.floor(jnp.clip(x * _SCALE + 4.0, 0.0, 7.5))
    q_ref[...] = t * _STEP - 1.0
    f_ref[...] = jnp.dot(
        t.astype(jnp.bfloat16), s_ref[...],
        preferred_element_type=jnp.float32).astype(jnp.int32)


def _tc_call(x2, sel):
    return pl.pallas_call(
        _tc_body,
        grid=(R_TC // BR,),
        in_specs=[
            pl.BlockSpec((BR, 256), lambda i: (i, 0)),
            pl.BlockSpec((256, 64), lambda i: (0, 0)),
        ],
        out_specs=[
            pl.BlockSpec((BR, 256), lambda i: (i, 0)),
            pl.BlockSpec((BR, 64), lambda i: (i, 0)),
        ],
        out_shape=[
            jax.ShapeDtypeStruct((R_TC, 256), jnp.float32),
            jax.ShapeDtypeStruct((R_TC, 64), jnp.int32),
        ],
    )(x2, sel)


_SEL = np.zeros((256, 64), dtype=np.float32)
for _d in range(256):
    _SEL[_d, _d // 4] = float((1, 8, 64, 512)[_d % 4])


@jax.jit
def kernel(latents):
    bsz, seq_len, dim = latents.shape
    x2 = latents.reshape(R_TOTAL, 256)
    xf = latents.reshape(-1)
    sel = jnp.asarray(_SEL, dtype=jnp.bfloat16)
    q_sc, f_sc = _sc_call(xf)
    q_tc, f_tc = _tc_call(x2, sel)
    q = jnp.concatenate([q_tc, q_sc.reshape(R_SC, 256)], axis=0)
    f = jnp.concatenate([f_tc, f_sc.reshape(R_SC, 64)], axis=0)
    return (
        q.reshape(bsz, seq_len, dim),
        f.reshape(bsz, seq_len, dim // 4),
    )


# speedup vs baseline: 1.0813x; 1.0813x over previous
import functools

import jax
import jax.numpy as jnp
import numpy as np
from jax import lax
from jax.experimental import pallas as pl
from jax.experimental.pallas import tpu as pltpu
from jax.experimental.pallas import tpu_sc as plsc

W = 32
R_TOTAL = 16384
R_SC = 4096
R_TC = R_TOTAL - R_SC
CHUNK = 16384
BIGROWS = R_TOTAL * 256 // CHUNK
SC_BIGROWS = R_SC * 256 // CHUNK
NCHUNK = SC_BIGROWS // W
BLK = CHUNK // 64

_SCALE = 3.5
_STEP = 2.0 / 7.0

BR = 2048
N_TC_BLOCKS = R_TC // BR
N_BLOCKS = R_TOTAL // BR


def _quantize_chunk(x_v, q_v, f_v):
    lane4 = lax.broadcasted_iota(jnp.int32, (16,), 0) * 4

    @plsc.parallel_loop(0, BLK, 1, unroll=8)
    def blk(i):
        i0 = lane4 + i * 64
        ids = []
        for j in range(4):
            ij = i0 + j
            x = plsc.load_gather(x_v, [ij])
            t = x * _SCALE + 4.0
            t = jnp.minimum(jnp.maximum(t, 0.0), 7.5)
            idx = t.astype(jnp.int32)
            q = idx.astype(jnp.float32) * _STEP - 1.0
            plsc.store_scatter(q_v, [ij], q)
            ids.append(idx)
        flat = ids[0] | (ids[1] << 3) | (ids[2] << 6) | (ids[3] << 9)
        f_v[pl.ds(i * 16, 16)] = flat


def _sc_body(x_hbm, q_hbm, f_hbm,
             x0, x1, q0, q1, f0, f1, si0, si1, so0, so1):
    wid = lax.axis_index("s") * 2 + lax.axis_index("c")
    xb, qb, fb = [x0, x1], [q0, q1], [f0, f1]
    si, so = [si0, si1], [so0, so1]
    in_copy = [None, None]
    out_q = [None, None]
    out_f = [None, None]

    base = (BIGROWS - SC_BIGROWS) + wid * NCHUNK
    in_copy[0] = pltpu.async_copy(x_hbm.at[base], xb[0], si[0])
    for c in range(NCHUNK):
        b = c & 1
        if c + 1 < NCHUNK:
            in_copy[1 - b] = pltpu.async_copy(
                x_hbm.at[base + c + 1], xb[1 - b], si[1 - b])
        in_copy[b].wait()
        if c >= 2:
            out_q[b].wait()
            out_f[b].wait()
        _quantize_chunk(xb[b], qb[b], fb[b])
        out_q[b] = pltpu.async_copy(
            qb[b], q_hbm.at[wid * NCHUNK + c], so[b])
        out_f[b] = pltpu.async_copy(
            fb[b], f_hbm.at[wid * NCHUNK + c], so[b])
    for b in range(min(2, NCHUNK)):
        out_q[b].wait()
        out_f[b].wait()


@functools.partial(
    pl.kernel,
    out_type=(
        jax.ShapeDtypeStruct((SC_BIGROWS, CHUNK), jnp.float32),
        jax.ShapeDtypeStruct((SC_BIGROWS, CHUNK // 4), jnp.int32),
    ),
    mesh=plsc.VectorSubcoreMesh(core_axis_name="c", subcore_axis_name="s"),
    scratch_types=(
        [pltpu.VMEM((CHUNK,), jnp.float32) for _ in range(4)]
        + [pltpu.VMEM((CHUNK // 4,), jnp.int32) for _ in range(2)]
        + [pltpu.SemaphoreType.DMA for _ in range(4)]
    ),
    compiler_params=pltpu.CompilerParams(needs_layout_passes=False),
)
def _sc_call(x_hbm, q_hbm, f_hbm, *bufs):
    _sc_body(x_hbm, q_hbm, f_hbm, *bufs)


def _tc_body(x_ref, qsc_ref, fsc_ref, s_ref, q_ref, f_ref):
    i = pl.program_id(0)

    @pl.when(i < N_TC_BLOCKS)
    def _compute():
        x = x_ref[...]
        t = jnp.floor(jnp.clip(x * _SCALE + 4.0, 0.0, 7.5))
        q_ref[...] = t * _STEP - 1.0
        f_ref[...] = jnp.dot(
            t.astype(jnp.bfloat16), s_ref[...],
            preferred_element_type=jnp.float32).astype(jnp.int32)

    @pl.when(i >= N_TC_BLOCKS)
    def _copy_sc():
        q_ref[...] = qsc_ref[...]
        f_ref[...] = fsc_ref[...]


def _tc_call(x2, q_sc, f_sc, sel):
    return pl.pallas_call(
        _tc_body,
        grid=(N_BLOCKS,),
        in_specs=[
            pl.BlockSpec((BR, 256), lambda i: (jnp.minimum(i, N_TC_BLOCKS - 1), 0)),
            pl.BlockSpec((BR, 256), lambda i: (jnp.maximum(i - N_TC_BLOCKS, 0), 0)),
            pl.BlockSpec((BR, 64), lambda i: (jnp.maximum(i - N_TC_BLOCKS, 0), 0)),
            pl.BlockSpec((256, 64), lambda i: (0, 0)),
        ],
        out_specs=[
            pl.BlockSpec((BR, 256), lambda i: (i, 0)),
            pl.BlockSpec((BR, 64), lambda i: (i, 0)),
        ],
        out_shape=[
            jax.ShapeDtypeStruct((R_TOTAL, 256), jnp.float32),
            jax.ShapeDtypeStruct((R_TOTAL, 64), jnp.int32),
        ],
    )(x2, q_sc, f_sc, sel)


_SEL = np.zeros((256, 64), dtype=np.float32)
for _d in range(256):
    _SEL[_d, _d // 4] = float((1, 8, 64, 512)[_d % 4])


@jax.jit
def kernel(latents):
    bsz, seq_len, dim = latents.shape
    x2 = latents.reshape(R_TOTAL, 256)
    x16 = latents.reshape(BIGROWS, CHUNK)
    sel = jnp.asarray(_SEL, dtype=jnp.bfloat16)
    q_sc, f_sc = _sc_call(x16)
    q, f = _tc_call(x2, q_sc.reshape(R_SC, 256), f_sc.reshape(R_SC, 64), sel)
    return (
        q.reshape(bsz, seq_len, dim),
        f.reshape(bsz, seq_len, dim // 4),
    )


# speedup vs baseline: 1.7509x; 1.6193x over previous
import functools

import jax
import jax.numpy as jnp
import numpy as np
from jax import lax
from jax.experimental import pallas as pl
from jax.experimental.pallas import tpu as pltpu
from jax.experimental.pallas import tpu_sc as plsc

W = 32
R_TOTAL = 16384
R_SC = 4096
R_TC = R_TOTAL - R_SC
SLAB = 64
NCHUNK = R_SC // SLAB // W
SEQ = 1024

_SCALE = 3.5
_STEP = 2.0 / 7.0

BR = 2048
N_TC_BLOCKS = R_TC // BR


def _quantize_chunk(x_v, q_v, f_v):
    lane4 = lax.broadcasted_iota(jnp.int32, (16,), 0) * 4
    zeros16 = jnp.zeros((16,), jnp.int32)

    @plsc.parallel_loop(0, SLAB, 1, unroll=2)
    def row_blk(r):
        rows = zeros16 + r
        for qtr in range(4):
            ids = []
            for j in range(4):
                cols = lane4 + (qtr * 64 + j)
                x = plsc.load_gather(x_v, [rows, cols])
                t = x * _SCALE + 4.0
                t = jnp.minimum(jnp.maximum(t, 0.0), 7.5)
                idx = t.astype(jnp.int32)
                q = idx.astype(jnp.float32) * _STEP - 1.0
                plsc.store_scatter(q_v, [rows, cols], q)
                ids.append(idx)
            flat = ids[0] | (ids[1] << 3) | (ids[2] << 6) | (ids[3] << 9)
            f_v[r, pl.ds(qtr * 16, 16)] = flat


def _sc_body(x_hbm, q_hbm, f_hbm,
             x0, x1, q0, q1, f0, f1, si0, si1, so0, so1):
    wid = lax.axis_index("s") * 2 + lax.axis_index("c")
    xb, qb, fb = [x0, x1], [q0, q1], [f0, f1]
    si, so = [si0, si1], [so0, so1]
    in_copy = [None, None]
    out_q = [None, None]
    out_f = [None, None]

    def x_slab(k):
        row = R_TC + k * SLAB
        b = row // SEQ
        r0 = pl.multiple_of(row % SEQ, SLAB)
        return x_hbm.at[b, pl.ds(r0, SLAB)]

    k0 = wid * NCHUNK
    in_copy[0] = pltpu.async_copy(x_slab(k0), xb[0], si[0])
    for c in range(NCHUNK):
        b = c & 1
        if c + 1 < NCHUNK:
            in_copy[1 - b] = pltpu.async_copy(
                x_slab(k0 + c + 1), xb[1 - b], si[1 - b])
        in_copy[b].wait()
        if c >= 2:
            out_q[b].wait()
            out_f[b].wait()
        _quantize_chunk(xb[b], qb[b], fb[b])
        off = pl.multiple_of((k0 + c) * SLAB, SLAB)
        out_q[b] = pltpu.async_copy(qb[b], q_hbm.at[pl.ds(off, SLAB)], so[b])
        out_f[b] = pltpu.async_copy(fb[b], f_hbm.at[pl.ds(off, SLAB)], so[b])
    for b in range(min(2, NCHUNK)):
        out_q[b].wait()
        out_f[b].wait()


@functools.partial(
    pl.kernel,
    out_type=(
        jax.ShapeDtypeStruct((R_SC, 256), jnp.float32),
        jax.ShapeDtypeStruct((R_SC, 64), jnp.int32),
    ),
    mesh=plsc.VectorSubcoreMesh(core_axis_name="c", subcore_axis_name="s"),
    scratch_types=(
        [pltpu.VMEM((SLAB, 256), jnp.float32) for _ in range(4)]
        + [pltpu.VMEM((SLAB, 64), jnp.int32) for _ in range(2)]
        + [pltpu.SemaphoreType.DMA for _ in range(4)]
    ),
    compiler_params=pltpu.CompilerParams(needs_layout_passes=False),
)
def _sc_call(x_hbm, q_hbm, f_hbm, *bufs):
    _sc_body(x_hbm, q_hbm, f_hbm, *bufs)


def _tc_body(x_ref, s_ref, q_ref, f_ref):
    x = x_ref[...]
    t = jnp.floor(jnp.clip(x * _SCALE + 4.0, 0.0, 7.5))
    q_ref[...] = t * _STEP - 1.0
    f_ref[...] = jnp.dot(
        t.astype(jnp.bfloat16), s_ref[...],
        preferred_element_type=jnp.float32).astype(jnp.int32)


def _tc_call(x2, sel):
    return pl.pallas_call(
        _tc_body,
        grid=(N_TC_BLOCKS,),
        in_specs=[
            pl.BlockSpec((BR, 256), lambda i: (i, 0)),
            pl.BlockSpec((256, 64), lambda i: (0, 0)),
        ],
        out_specs=[
            pl.BlockSpec((BR, 256), lambda i: (i, 0)),
            pl.BlockSpec((BR, 64), lambda i: (i, 0)),
        ],
        out_shape=[
            jax.ShapeDtypeStruct((R_TOTAL, 256), jnp.float32),
            jax.ShapeDtypeStruct((R_TOTAL, 64), jnp.int32),
        ],
    )(x2, sel)


_SEL = np.zeros((256, 64), dtype=np.float32)
for _d in range(256):
    _SEL[_d, _d // 4] = float((1, 8, 64, 512)[_d % 4])


@jax.jit
def kernel(latents):
    bsz, seq_len, dim = latents.shape
    x2 = latents.reshape(R_TOTAL, 256)
    sel = jnp.asarray(_SEL, dtype=jnp.bfloat16)
    q_sc, f_sc = _sc_call(latents)
    q_tc, f_tc = _tc_call(x2, sel)
    q = lax.dynamic_update_slice(q_tc, q_sc, (R_TC, 0))
    f = lax.dynamic_update_slice(f_tc, f_sc, (R_TC, 0))
    return (
        q.reshape(bsz, seq_len, dim),
        f.reshape(bsz, seq_len, dim // 4),
    )
